# baseline (speedup 1.0000x reference)
import jax
import jax.numpy as jnp
from jax import lax
from jax.experimental import pallas as pl
from jax.experimental.pallas import tpu as pltpu


def _vq_block(z_ref, ets_ref, rhs_ref, idx_ref, emb_ref, loss_ref, esq_ref):
    i = pl.program_id(0)
    K = ets_ref.shape[1]
    D = z_ref.shape[1]

    @pl.when(i == 0)
    def _():
        et2 = ets_ref[...]
        esq_ref[...] = 0.25 * jnp.sum(et2 * et2, axis=0, keepdims=True)
        loss_ref[...] = jnp.zeros_like(loss_ref)

    zb = z_ref[...]
    dot = jnp.dot(zb, ets_ref[...], preferred_element_type=jnp.float32)
    zsq = jnp.sum(zb * zb, axis=1, keepdims=True)
    dist = (zsq + dot) + esq_ref[...]
    minv = jnp.min(dist, axis=1, keepdims=True)
    eq = dist == minv
    onehot = jnp.where(eq, jnp.float32(1), jnp.float32(0))
    aug = jnp.dot(onehot, rhs_ref[...],
                  preferred_element_type=jnp.float32)
    emb_ref[...] = aug[:, :D]
    hi = aug[:, D:D + 1]
    lo = aug[:, D + 1:D + 2]
    cnt = aug[:, D + 2:D + 3]
    idx_ref[...] = (128.0 * hi + lo).astype(jnp.int32)
    loss_ref[...] += jnp.sum(minv).reshape(1, 1)

    @pl.when(jnp.max(cnt) > 1.5)
    def _():
        iota = lax.broadcasted_iota(jnp.int32, (1, K), 1).astype(jnp.float32)
        masked = jnp.where(eq, iota, jnp.float32(K))
        idxf = jnp.min(masked, axis=1, keepdims=True)
        oh2 = jnp.where(masked == idxf, jnp.float32(1), jnp.float32(0))
        emb_ref[...] = jnp.dot(oh2, rhs_ref[...],
                               preferred_element_type=jnp.float32)[:, :D]
        idx_ref[...] = idxf.astype(jnp.int32)


def kernel(z, embeddings):
    B, T, D = z.shape
    N = B * T
    K = embeddings.shape[0]
    BLK = 3072
    NB = N // BLK
    flat = z.reshape(N, D)
    ets = -2.0 * embeddings.T
    codes = jnp.arange(K, dtype=jnp.float32)
    rhs = jnp.concatenate(
        [embeddings,
         (codes // 128)[:, None],
         (codes % 128)[:, None],
         jnp.ones((K, 1), jnp.float32),
         jnp.zeros((K, 125), jnp.float32)], axis=1)

    idx_col, emb, loss_sum = pl.pallas_call(
        _vq_block,
        grid=(NB,),
        in_specs=[
            pl.BlockSpec((BLK, D), lambda i: (i, 0)),
            pl.BlockSpec((D, K), lambda i: (0, 0)),
            pl.BlockSpec((K, D + 128), lambda i: (0, 0)),
        ],
        out_specs=[
            pl.BlockSpec((BLK, 1), lambda i: (i, 0)),
            pl.BlockSpec((BLK, D), lambda i: (i, 0)),
            pl.BlockSpec((1, 1), lambda i: (0, 0)),
        ],
        out_shape=[
            jax.ShapeDtypeStruct((N, 1), jnp.int32),
            jax.ShapeDtypeStruct((N, D), jnp.float32),
            jax.ShapeDtypeStruct((1, 1), jnp.float32),
        ],
        scratch_shapes=[pltpu.VMEM((1, K), jnp.float32)],
    )(flat, ets, rhs)

    encoding_indices = idx_col.reshape(B, T)
    emb = emb.reshape(B, T, D)
    commitment_loss = 0.25 * loss_sum[0, 0] / (N * D)
    return emb, encoding_indices, commitment_loss


# speedup vs baseline: 1.0087x; 1.0087x over previous
import jax
import jax.numpy as jnp
from jax import lax
from jax.experimental import pallas as pl
from jax.experimental.pallas import tpu as pltpu


def _vq_block(z_ref, ets_ref, rhs_ref, idx_ref, emb_ref, loss_ref, esq_ref):
    i = pl.program_id(0)
    K = ets_ref.shape[1]
    D = z_ref.shape[1]

    @pl.when(i == 0)
    def _():
        et2 = ets_ref[...]
        esq_ref[...] = 0.25 * jnp.sum(et2 * et2, axis=0, keepdims=True)
        loss_ref[...] = jnp.zeros_like(loss_ref)

    zb = z_ref[...]
    dot = jnp.dot(zb, ets_ref[...], preferred_element_type=jnp.float32)
    zsq = jnp.sum(zb * zb, axis=1, keepdims=True)
    dist = (zsq + dot) + esq_ref[...]
    minv = jnp.min(dist, axis=1, keepdims=True)
    eq = dist == minv
    onehot = jnp.where(eq, jnp.float32(1), jnp.float32(0))
    aug = jnp.dot(onehot, rhs_ref[...],
                  preferred_element_type=jnp.float32)
    emb_ref[...] = aug[:, :D]
    hi = aug[:, D:D + 1]
    lo = aug[:, D + 1:D + 2]
    cnt = aug[:, D + 2:D + 3]
    idx_ref[...] = (128.0 * hi + lo).astype(jnp.int32)
    loss_ref[...] += jnp.sum(minv).reshape(1, 1)

    @pl.when(jnp.max(cnt) > 1.5)
    def _():
        iota = lax.broadcasted_iota(jnp.int32, (1, K), 1).astype(jnp.float32)
        masked = jnp.where(eq, iota, jnp.float32(K))
        idxf = jnp.min(masked, axis=1, keepdims=True)
        oh2 = jnp.where(masked == idxf, jnp.float32(1), jnp.float32(0))
        emb_ref[...] = jnp.dot(oh2, rhs_ref[...],
                               preferred_element_type=jnp.float32)[:, :D]
        idx_ref[...] = idxf.astype(jnp.int32)


def kernel(z, embeddings):
    B, T, D = z.shape
    N = B * T
    K = embeddings.shape[0]
    BLK = 2304
    NB = N // BLK
    flat = z.reshape(N, D)
    ets = -2.0 * embeddings.T
    codes = jnp.arange(K, dtype=jnp.float32)
    rhs = jnp.concatenate(
        [embeddings,
         (codes // 128)[:, None],
         (codes % 128)[:, None],
         jnp.ones((K, 1), jnp.float32),
         jnp.zeros((K, 125), jnp.float32)], axis=1)

    idx_col, emb, loss_sum = pl.pallas_call(
        _vq_block,
        grid=(NB,),
        in_specs=[
            pl.BlockSpec((BLK, D), lambda i: (i, 0)),
            pl.BlockSpec((D, K), lambda i: (0, 0)),
            pl.BlockSpec((K, D + 128), lambda i: (0, 0)),
        ],
        out_specs=[
            pl.BlockSpec((BLK, 1), lambda i: (i, 0)),
            pl.BlockSpec((BLK, D), lambda i: (i, 0)),
            pl.BlockSpec((1, 1), lambda i: (0, 0)),
        ],
        out_shape=[
            jax.ShapeDtypeStruct((N, 1), jnp.int32),
            jax.ShapeDtypeStruct((N, D), jnp.float32),
            jax.ShapeDtypeStruct((1, 1), jnp.float32),
        ],
        scratch_shapes=[pltpu.VMEM((1, K), jnp.float32)],
    )(flat, ets, rhs)

    encoding_indices = idx_col.reshape(B, T)
    emb = emb.reshape(B, T, D)
    commitment_loss = 0.25 * loss_sum[0, 0] / (N * D)
    return emb, encoding_indices, commitment_loss


# speedup vs baseline: 1.2480x; 1.2372x over previous
import jax
import jax.numpy as jnp
from jax import lax
from jax.experimental import pallas as pl
from jax.experimental.pallas import tpu as pltpu


def _vq_block(z_ref, e_ref, idx_ref, emb_ref, loss_ref, ets_s, rhs_s, esq_s):
    i = pl.program_id(0)
    K, D = e_ref.shape
    BPB, T, _ = z_ref.shape
    BLK = BPB * T

    @pl.when(i == 0)
    def _():
        e = e_ref[...]
        ets = -2.0 * jnp.transpose(e)
        ets_s[...] = ets
        esq_s[...] = 0.25 * jnp.sum(ets * ets, axis=0, keepdims=True)
        rhs_s[:, :D] = e
        r = lax.broadcasted_iota(jnp.int32, (K, 128), 0)
        lane = lax.broadcasted_iota(jnp.int32, (K, 128), 1)
        hi = (r // 128).astype(jnp.float32)
        lo = (r % 128).astype(jnp.float32)
        extras = jnp.where(
            lane == 0, hi,
            jnp.where(lane == 1, lo,
                      jnp.where(lane == 2, jnp.float32(1), jnp.float32(0))))
        rhs_s[:, D:D + 128] = extras
        loss_ref[...] = jnp.zeros_like(loss_ref)

    zb = z_ref[...].reshape(BLK, D)
    dot = jnp.dot(zb, ets_s[...], preferred_element_type=jnp.float32)
    zsq = jnp.sum(zb * zb, axis=1, keepdims=True)
    dist = (zsq + dot) + esq_s[...]
    minv = jnp.min(dist, axis=1, keepdims=True)
    eq = dist == minv
    onehot = jnp.where(eq, jnp.float32(1), jnp.float32(0))
    aug = jnp.dot(onehot, rhs_s[...],
                  preferred_element_type=jnp.float32)
    emb_ref[...] = aug[:, :D].reshape(BPB, T, D)
    hi = aug[:, D:D + 1]
    lo = aug[:, D + 1:D + 2]
    cnt = aug[:, D + 2:D + 3]
    idx_ref[...] = (128.0 * hi + lo).astype(jnp.int32)
    loss_ref[...] += jnp.sum(minv).reshape(1, 1)

    @pl.when(jnp.max(cnt) > 1.5)
    def _():
        iota = lax.broadcasted_iota(jnp.int32, (1, K), 1).astype(jnp.float32)
        masked = jnp.where(eq, iota, jnp.float32(K))
        idxf = jnp.min(masked, axis=1, keepdims=True)
        oh2 = jnp.where(masked == idxf, jnp.float32(1), jnp.float32(0))
        emb_ref[...] = jnp.dot(
            oh2, rhs_s[...],
            preferred_element_type=jnp.float32)[:, :D].reshape(BPB, T, D)
        idx_ref[...] = idxf.astype(jnp.int32)


def kernel(z, embeddings):
    B, T, D = z.shape
    N = B * T
    K = embeddings.shape[0]
    BLK = 2304
    BPB = BLK // T
    NB = N // BLK

    idx_col, emb, loss_sum = pl.pallas_call(
        _vq_block,
        grid=(NB,),
        in_specs=[
            pl.BlockSpec((BPB, T, D), lambda i: (i, 0, 0)),
            pl.BlockSpec((K, D), lambda i: (0, 0)),
        ],
        out_specs=[
            pl.BlockSpec((BLK, 1), lambda i: (i, 0)),
            pl.BlockSpec((BPB, T, D), lambda i: (i, 0, 0)),
            pl.BlockSpec((1, 1), lambda i: (0, 0)),
        ],
        out_shape=[
            jax.ShapeDtypeStruct((N, 1), jnp.int32),
            jax.ShapeDtypeStruct((B, T, D), jnp.float32),
            jax.ShapeDtypeStruct((1, 1), jnp.float32),
        ],
        scratch_shapes=[
            pltpu.VMEM((D, K), jnp.float32),
            pltpu.VMEM((K, D + 128), jnp.float32),
            pltpu.VMEM((1, K), jnp.float32),
        ],
    )(z, embeddings)

    encoding_indices = idx_col.reshape(B, T)
    commitment_loss = 0.25 * loss_sum[0, 0] / (N * D)
    return emb, encoding_indices, commitment_loss
